# baseline (speedup 1.0000x reference)
import jax
import jax.numpy as jnp
from jax import lax
from jax.experimental import pallas as pl
from jax.experimental.pallas import tpu as pltpu
from jax.experimental.pallas import tpu_sc as plsc

_VOCAB = 1000000
_D = 128
_CTX = 200
_HID = 128

_PER_W = 8
_ACTIVE = _CTX // _PER_W

_T = 32768
_NSTEPS = (_VOCAB + _T - 1) // _T

_C = 4000
_NCH = _VOCAB // _C
_K = 6
_KO = 4


def _sc_body(idx_hbm, emb_hbm, out_hbm, idx_v, rows_v, part_v, sem):
    wid = lax.axis_index("s") * 2 + lax.axis_index("c")

    @pl.when(wid < _ACTIVE)
    def _():
        base = wid * _PER_W
        pltpu.sync_copy(idx_hbm.at[pl.ds(base, _PER_W)], idx_v)
        pltpu.async_copy(emb_hbm.at[idx_v], rows_v, sem).wait()
        for c in range(_D // 16):
            acc = jnp.zeros((16,), jnp.float32)
            for r in range(_PER_W):
                acc = acc + rows_v[r, pl.ds(c * 16, 16)]
            part_v[pl.ds(c * 16, 16)] = acc
        pltpu.sync_copy(part_v, out_hbm.at[wid])


def _sc_gather_sum(x, emb):
    f = pl.kernel(
        _sc_body,
        out_type=jax.ShapeDtypeStruct((_ACTIVE, _D), jnp.float32),
        mesh=plsc.VectorSubcoreMesh(core_axis_name="c", subcore_axis_name="s"),
        scratch_types=[
            pltpu.VMEM((_PER_W,), jnp.int32),
            pltpu.VMEM((_PER_W, _D), jnp.float32),
            pltpu.VMEM((_D,), jnp.float32),
            pltpu.SemaphoreType.DMA,
        ],
    )
    return f(x, emb)


def _logits_body(parts, wp, w1, b1, w2_hbm, b2_hbm, o_hbm, lse_ref,
                 w2buf, b2buf, obuf, in_sems, ob_sems, out_sems,
                 m_scr, s_scr):
    s = jnp.sum(parts[...], axis=0, keepdims=True)
    p = lax.dot_general(s, wp[...], (((1,), (1,)), ((), ())),
                        preferred_element_type=jnp.float32)
    h = lax.dot_general(p, w1[...], (((1,), (1,)), ((), ())),
                        preferred_element_type=jnp.float32) + b1[...]
    h = jnp.maximum(h, 0.0).astype(jnp.bfloat16)
    m_scr[0] = -jnp.inf
    s_scr[0] = 0.0

    def _start_in(c, slot):
        pltpu.make_async_copy(
            w2_hbm.at[pl.ds(c * _C, _C), :], w2buf.at[slot],
            in_sems.at[slot]).start()

    b2_cp = pltpu.make_async_copy(b2_hbm, b2buf, ob_sems)
    b2_cp.start()
    for c in range(_K - 1):
        _start_in(c, c)
    b2_cp.wait()

    def _step(i, _):
        slot = lax.rem(i, _K)
        nxt = i + _K - 1

        @pl.when(nxt < _NCH)
        def _():
            _start_in(nxt, lax.rem(nxt, _K))

        pltpu.make_async_copy(
            w2_hbm.at[pl.ds(i * _C, _C), :], w2buf.at[slot],
            in_sems.at[slot]).wait()

        o_t = lax.dot_general(h, w2buf[slot].astype(jnp.bfloat16),
                              (((1,), (1,)), ((), ())),
                              preferred_element_type=jnp.float32) + b2buf[pl.ds(i, 1), :]
        m_old = m_scr[0]
        m_new = jnp.maximum(m_old, jnp.max(o_t))
        s_scr[0] = s_scr[0] * jnp.exp(m_old - m_new) + jnp.sum(jnp.exp(o_t - m_new))
        m_scr[0] = m_new

        oslot = lax.rem(i, _KO)

        @pl.when(i >= _KO)
        def _():
            pltpu.make_async_copy(
                obuf.at[oslot], o_hbm.at[pl.ds(i - _KO, 1), :],
                out_sems.at[oslot]).wait()

        obuf[oslot] = o_t
        pltpu.make_async_copy(
            obuf.at[oslot], o_hbm.at[pl.ds(i, 1), :],
            out_sems.at[oslot]).start()
        return 0

    lax.fori_loop(0, _NCH, _step, 0)

    for d in range(_KO):
        c = _NCH - _KO + d
        pltpu.make_async_copy(
            obuf.at[c % _KO], o_hbm.at[pl.ds(c, 1), :],
            out_sems.at[c % _KO]).wait()

    lse_ref[0, 0] = m_scr[0] + jnp.log(s_scr[0])


def _tc_logits(parts, wp, w1, b1, w2, b2):
    return pl.pallas_call(
        _logits_body,
        in_specs=[
            pl.BlockSpec(memory_space=pltpu.VMEM),
            pl.BlockSpec(memory_space=pltpu.VMEM),
            pl.BlockSpec(memory_space=pltpu.VMEM),
            pl.BlockSpec(memory_space=pltpu.VMEM),
            pl.BlockSpec(memory_space=pl.ANY),
            pl.BlockSpec(memory_space=pl.ANY),
        ],
        out_specs=[
            pl.BlockSpec(memory_space=pl.ANY),
            pl.BlockSpec(memory_space=pltpu.SMEM),
        ],
        out_shape=[
            jax.ShapeDtypeStruct((_NCH, _C), jnp.float32),
            jax.ShapeDtypeStruct((1, 1), jnp.float32),
        ],
        scratch_shapes=[
            pltpu.VMEM((_K, _C, _D), jnp.float32),
            pltpu.VMEM((_NCH, _C), jnp.float32),
            pltpu.VMEM((_KO, 1, _C), jnp.float32),
            pltpu.SemaphoreType.DMA((_K,)),
            pltpu.SemaphoreType.DMA,
            pltpu.SemaphoreType.DMA((_KO,)),
            pltpu.SMEM((1,), jnp.float32),
            pltpu.SMEM((1,), jnp.float32),
        ],
    )(parts, wp, w1, b1, w2, b2)


def _norm_step(o_ref, lse_ref, out_ref):
    out_ref[...] = o_ref[...] - lse_ref[0, 0]


def _tc_norm(o, lse):
    return pl.pallas_call(
        _norm_step,
        grid=(_NSTEPS,),
        in_specs=[
            pl.BlockSpec((1, _T), lambda i: (0, i)),
            pl.BlockSpec(memory_space=pltpu.SMEM),
        ],
        out_specs=pl.BlockSpec((1, _T), lambda i: (0, i)),
        out_shape=jax.ShapeDtypeStruct((1, _VOCAB), jnp.float32),
    )(o, lse)


def kernel(x, emb, W_proj, W1, b1, W2, b2):
    x = x.astype(jnp.int32)
    parts = _sc_gather_sum(x, emb)
    o2, lse = _tc_logits(parts, W_proj, W1,
                         b1.reshape(1, _HID), W2, b2.reshape(_NCH, _C))
    return _tc_norm(o2.reshape(1, _VOCAB), lse)


# speedup vs baseline: 1.1470x; 1.1470x over previous
import jax
import jax.numpy as jnp
from jax import lax
from jax.experimental import pallas as pl
from jax.experimental.pallas import tpu as pltpu
from jax.experimental.pallas import tpu_sc as plsc

_VOCAB = 1000000
_D = 128
_CTX = 200
_HID = 128

_PER_W = 8
_ACTIVE = _CTX // _PER_W

_T = 32768
_NSTEPS = (_VOCAB + _T - 1) // _T

_C = 8000
_NCH = _VOCAB // _C
_K = 4
_KO = 4


def _sc_body(idx_hbm, emb_hbm, out_hbm, idx_v, rows_v, part_v, sem):
    wid = lax.axis_index("s") * 2 + lax.axis_index("c")

    @pl.when(wid < _ACTIVE)
    def _():
        base = wid * _PER_W
        pltpu.sync_copy(idx_hbm.at[pl.ds(base, _PER_W)], idx_v)
        pltpu.async_copy(emb_hbm.at[idx_v], rows_v, sem).wait()
        for c in range(_D // 16):
            acc = jnp.zeros((16,), jnp.float32)
            for r in range(_PER_W):
                acc = acc + rows_v[r, pl.ds(c * 16, 16)]
            part_v[pl.ds(c * 16, 16)] = acc
        pltpu.sync_copy(part_v, out_hbm.at[wid])


def _sc_gather_sum(x, emb):
    f = pl.kernel(
        _sc_body,
        out_type=jax.ShapeDtypeStruct((_ACTIVE, _D), jnp.float32),
        mesh=plsc.VectorSubcoreMesh(core_axis_name="c", subcore_axis_name="s"),
        scratch_types=[
            pltpu.VMEM((_PER_W,), jnp.int32),
            pltpu.VMEM((_PER_W, _D), jnp.float32),
            pltpu.VMEM((_D,), jnp.float32),
            pltpu.SemaphoreType.DMA,
        ],
    )
    return f(x, emb)


def _logits_body(parts, wp, w1, b1, w2_hbm, b2_hbm, o_hbm, lse_ref,
                 w2buf, b2buf, obuf, in_sems, ob_sems, out_sems,
                 m_scr, s_scr):
    s = jnp.sum(parts[...], axis=0, keepdims=True)
    p = lax.dot_general(s, wp[...], (((1,), (1,)), ((), ())),
                        preferred_element_type=jnp.float32)
    h = lax.dot_general(p, w1[...], (((1,), (1,)), ((), ())),
                        preferred_element_type=jnp.float32) + b1[...]
    h = jnp.maximum(h, 0.0).astype(jnp.bfloat16)
    m_scr[0] = -jnp.inf
    s_scr[0] = 0.0

    def _start_in(c, slot):
        pltpu.make_async_copy(
            w2_hbm.at[pl.ds(c * _C, _C), :], w2buf.at[slot],
            in_sems.at[slot]).start()

    b2_cp = pltpu.make_async_copy(b2_hbm, b2buf, ob_sems)
    b2_cp.start()
    for c in range(_K - 1):
        _start_in(c, c)
    b2_cp.wait()

    def _step(i, _):
        slot = lax.rem(i, _K)
        nxt = i + _K - 1

        @pl.when(nxt < _NCH)
        def _():
            _start_in(nxt, lax.rem(nxt, _K))

        pltpu.make_async_copy(
            w2_hbm.at[pl.ds(i * _C, _C), :], w2buf.at[slot],
            in_sems.at[slot]).wait()

        o_t = lax.dot_general(h, w2buf[slot].astype(jnp.bfloat16),
                              (((1,), (1,)), ((), ())),
                              preferred_element_type=jnp.float32) + b2buf[pl.ds(i, 1), :]
        m_old = m_scr[0]
        m_new = jnp.maximum(m_old, jnp.max(o_t))
        s_scr[0] = s_scr[0] * jnp.exp(m_old - m_new) + jnp.sum(jnp.exp(o_t - m_new))
        m_scr[0] = m_new

        oslot = lax.rem(i, _KO)

        @pl.when(i >= _KO)
        def _():
            pltpu.make_async_copy(
                obuf.at[oslot], o_hbm.at[pl.ds(i - _KO, 1), :],
                out_sems.at[oslot]).wait()

        obuf[oslot] = o_t
        pltpu.make_async_copy(
            obuf.at[oslot], o_hbm.at[pl.ds(i, 1), :],
            out_sems.at[oslot]).start()
        return 0

    lax.fori_loop(0, _NCH, _step, 0)

    for d in range(_KO):
        c = _NCH - _KO + d
        pltpu.make_async_copy(
            obuf.at[c % _KO], o_hbm.at[pl.ds(c, 1), :],
            out_sems.at[c % _KO]).wait()

    lse_ref[0, 0] = m_scr[0] + jnp.log(s_scr[0])


def _tc_logits(parts, wp, w1, b1, w2, b2):
    return pl.pallas_call(
        _logits_body,
        in_specs=[
            pl.BlockSpec(memory_space=pltpu.VMEM),
            pl.BlockSpec(memory_space=pltpu.VMEM),
            pl.BlockSpec(memory_space=pltpu.VMEM),
            pl.BlockSpec(memory_space=pltpu.VMEM),
            pl.BlockSpec(memory_space=pl.ANY),
            pl.BlockSpec(memory_space=pl.ANY),
        ],
        out_specs=[
            pl.BlockSpec(memory_space=pl.ANY),
            pl.BlockSpec(memory_space=pltpu.SMEM),
        ],
        out_shape=[
            jax.ShapeDtypeStruct((_NCH, _C), jnp.float32),
            jax.ShapeDtypeStruct((1, 1), jnp.float32),
        ],
        scratch_shapes=[
            pltpu.VMEM((_K, _C, _D), jnp.float32),
            pltpu.VMEM((_NCH, _C), jnp.float32),
            pltpu.VMEM((_KO, 1, _C), jnp.float32),
            pltpu.SemaphoreType.DMA((_K,)),
            pltpu.SemaphoreType.DMA,
            pltpu.SemaphoreType.DMA((_KO,)),
            pltpu.SMEM((1,), jnp.float32),
            pltpu.SMEM((1,), jnp.float32),
        ],
    )(parts, wp, w1, b1, w2, b2)


def _norm_step(o_ref, lse_ref, out_ref):
    out_ref[...] = o_ref[...] - lse_ref[0, 0]


def _tc_norm(o, lse):
    return pl.pallas_call(
        _norm_step,
        grid=(_NSTEPS,),
        in_specs=[
            pl.BlockSpec((1, _T), lambda i: (0, i)),
            pl.BlockSpec(memory_space=pltpu.SMEM),
        ],
        out_specs=pl.BlockSpec((1, _T), lambda i: (0, i)),
        out_shape=jax.ShapeDtypeStruct((1, _VOCAB), jnp.float32),
    )(o, lse)


def kernel(x, emb, W_proj, W1, b1, W2, b2):
    x = x.astype(jnp.int32)
    parts = _sc_gather_sum(x, emb)
    o2, lse = _tc_logits(parts, W_proj, W1,
                         b1.reshape(1, _HID), W2, b2.reshape(_NCH, _C))
    return _tc_norm(o2.reshape(1, _VOCAB), lse)
